# baseline (speedup 1.0000x reference)
import functools

import jax
import jax.numpy as jnp
from jax import lax
from jax.experimental import pallas as pl
from jax.experimental.pallas import tpu as pltpu
from jax.experimental.pallas import tpu_sc as plsc

B, S, D = 16384, 200, 64
NC, NS = 2, 16
NW = NC * NS
ROWS_W = B // NW
CR = 2
NCHUNK = ROWS_W // CR

_mesh = plsc.VectorSubcoreMesh(core_axis_name="c", subcore_axis_name="s")


@functools.partial(
    pl.kernel,
    mesh=_mesh,
    out_type=jax.ShapeDtypeStruct((B, S, D), jnp.float32),
    scratch_types=[
        pltpu.VMEM((CR, S), jnp.int32),
        pltpu.VMEM((CR, S, D), jnp.float32),
        pltpu.VMEM((CR, S, D), jnp.float32),
        pltpu.VMEM((2 * D,), jnp.float32),
        pltpu.SemaphoreType.DMA,
        pltpu.SemaphoreType.DMA,
    ],
)
def _sc_lookup(idx_hbm, t_hbm, out_hbm, idx_v, out_v0, out_v1, t_v, sem0, sem1):
    wid = lax.axis_index("s") * NC + lax.axis_index("c")
    slab = wid * ROWS_W
    pltpu.sync_copy(t_hbm, t_v)
    t0 = [t_v[pl.ds(g * 16, 16)] for g in range(4)]
    dd = [t_v[pl.ds(D + g * 16, 16)] - t0[g] for g in range(4)]
    bufs = (out_v0, out_v1)
    sems = (sem0, sem1)

    def compute_chunk(k, buf):
        rowbase = slab + k * CR
        pltpu.sync_copy(idx_hbm.at[pl.ds(rowbase, CR)], idx_v)

        def row_body(r, carry2):
            def emit_col(j, f_lane):
                for g in range(4):
                    buf[r, j, pl.ds(g * 16, 16)] = t0[g] + f_lane * dd[g]

            for jb in range(S // 16):
                vf = idx_v[r, pl.ds(jb * 16, 16)].astype(jnp.float32)
                for l in range(16):
                    emit_col(jb * 16 + l, vf[l])
            vf = idx_v[r, pl.ds(S - 16, 16)].astype(jnp.float32)
            for l in range(8, 16):
                emit_col(S - 16 + l, vf[l])
            return carry2

        lax.fori_loop(0, CR, row_body, 0)

    def pair_body(kk, carry):
        for ph in range(2):
            k = kk * 2 + ph
            rowbase = slab + k * CR

            @pl.when(kk >= 1)
            def _wait():
                pltpu.make_async_copy(
                    bufs[ph], out_hbm.at[pl.ds(rowbase - 2 * CR, CR)], sems[ph]
                ).wait()

            compute_chunk(k, bufs[ph])
            pltpu.make_async_copy(
                bufs[ph], out_hbm.at[pl.ds(rowbase, CR)], sems[ph]
            ).start()
        return carry

    lax.fori_loop(0, NCHUNK // 2, pair_body, 0)
    for ph in range(2):
        k_last = NCHUNK - 2 + ph
        pltpu.make_async_copy(
            bufs[ph], out_hbm.at[pl.ds(slab + k_last * CR, CR)], sems[ph]
        ).wait()


def kernel(is_controller, table):
    idx = is_controller.astype(jnp.int32)
    tflat = table.reshape(2 * D)
    return _sc_lookup(idx, tflat)


# speedup vs baseline: 1.4631x; 1.4631x over previous
import jax
import jax.numpy as jnp
from jax.experimental import pallas as pl

B, S, D = 16384, 200, 64
P = S // 2
ROWS = 256


def _body(ia_ref, ib_ref, t_ref, out_ref):
    a = ia_ref[...].astype(jnp.float32)[:, :, None]
    b = ib_ref[...].astype(jnp.float32)[:, :, None]
    t = t_ref[...]
    t0 = t[0, :]
    d = t[1, :]
    lane = jax.lax.broadcasted_iota(jnp.int32, (ROWS, P, 2 * D), 2)
    f = jnp.where(lane < D, jnp.broadcast_to(a, (ROWS, P, 2 * D)),
                  jnp.broadcast_to(b, (ROWS, P, 2 * D)))
    out_ref[...] = t0[None, None, :] + f * d[None, None, :]


def kernel(is_controller, table):
    idx = is_controller.astype(jnp.int32)
    idx3 = idx.reshape(B, P, 2)
    ia = idx3[:, :, 0]
    ib = idx3[:, :, 1]
    t0 = table[0, :]
    d = table[1, :] - table[0, :]
    taux = jnp.stack([jnp.concatenate([t0, t0]), jnp.concatenate([d, d])])
    out = pl.pallas_call(
        _body,
        grid=(B // ROWS,),
        in_specs=[
            pl.BlockSpec((ROWS, P), lambda i: (i, 0)),
            pl.BlockSpec((ROWS, P), lambda i: (i, 0)),
            pl.BlockSpec((2, 2 * D), lambda i: (0, 0)),
        ],
        out_specs=pl.BlockSpec((ROWS, P, 2 * D), lambda i: (i, 0, 0)),
        out_shape=jax.ShapeDtypeStruct((B, P, 2 * D), jnp.float32),
    )(ia, ib, taux)
    return out.reshape(B, S, D)
